# baseline (speedup 1.0000x reference)
import functools

import jax
from jax.experimental.compute_on import compute_on
import jax.numpy as jnp
from jax import lax
from jax.experimental import pallas as pl
from jax.experimental.pallas import tpu as pltpu
from jax.experimental.pallas import tpu_sc as plsc

_NC = 2
_NS = 16
_NW = _NC * _NS
_CH = 256
_G = 8


def _tc_body(idx_ref, c_ref, x_ref, o_ref):
    idx = idx_ref[0]
    o_ref[...] = c_ref[...]
    o_ref[:, pl.ds(idx, 1), :] = x_ref[...]


def _tc_update(cache3, x3, idx):
    N, S, D = cache3.shape
    cache_spec = pl.BlockSpec((_G, S, D), lambda i, idx_ref: (i, 0, 0))
    x_spec = pl.BlockSpec((_G, 1, D), lambda i, idx_ref: (i, 0, 0))
    grid_spec = pltpu.PrefetchScalarGridSpec(
        num_scalar_prefetch=1,
        grid=(N // _G,),
        in_specs=[cache_spec, x_spec],
        out_specs=cache_spec,
    )
    return pl.pallas_call(
        _tc_body,
        grid_spec=grid_spec,
        out_shape=jax.ShapeDtypeStruct(cache3.shape, cache3.dtype),
    )(idx, cache3, x3)


def _sc_update(cache2, x2, idxv, N, S, D):
    slabs_per_w = N // _NW
    rows_per_core = N // _NC
    mesh = plsc.VectorSubcoreMesh(
        core_axis_name="c", subcore_axis_name="s", num_cores=_NC, num_subcores=_NS
    )

    @functools.partial(
        pl.kernel,
        out_type=jax.ShapeDtypeStruct((N * S, D), cache2.dtype),
        mesh=mesh,
        compiler_params=pltpu.CompilerParams(skip_device_barrier=True),
        scratch_types=[
            pltpu.VMEM((16,), jnp.int32),
            pltpu.VMEM((rows_per_core,), jnp.int32),
            pltpu.VMEM((_CH, D), jnp.float32),
            pltpu.VMEM((_CH, D), jnp.float32),
            pltpu.VMEM((_CH, D), jnp.float32),
            pltpu.SemaphoreType.DMA,
            pltpu.SemaphoreType.DMA,
            pltpu.SemaphoreType.DMA,
            pltpu.SemaphoreType.DMA,
            pltpu.SemaphoreType.DMA,
            pltpu.SemaphoreType.DMA,
            pltpu.SemaphoreType.DMA,
        ],
    )
    def sc_kernel(
        c_hbm, x_hbm, idx_hbm, o_hbm,
        idx_v, rowid_v, buf0, buf1, buf2,
        sin0, sin1, sin2, sout0, sout1, sout2, sem_s,
    ):
        c = lax.axis_index("c")
        s = lax.axis_index("s")
        wid = c * _NS + s
        base_slab = wid * slabs_per_w
        bufs = (buf0, buf1, buf2)
        sins = (sin0, sin1, sin2)
        souts = (sout0, sout1, sout2)
        nbuf = len(bufs)
        chunks_per_slab = S // _CH
        chunk_list = []
        for j in range(slabs_per_w):
            for p in range(chunks_per_slab):
                chunk_list.append((base_slab + j) * S + p * _CH)
        total = len(chunk_list)
        incopies = [None] * nbuf
        outcopies = [None] * nbuf
        for t in range(total + 1):
            if t < total:
                b = t % nbuf
                if t >= nbuf:
                    outcopies[b].wait()
                    outcopies[b] = None
                cp = pltpu.make_async_copy(
                    c_hbm.at[pl.ds(chunk_list[t], _CH)], bufs[b], sins[b]
                )
                cp.start()
                incopies[b] = cp
            if t >= 1:
                p = (t - 1) % nbuf
                incopies[p].wait()
                ocp = pltpu.make_async_copy(
                    bufs[p], o_hbm.at[pl.ds(chunk_list[t - 1], _CH)], souts[p]
                )
                ocp.start()
                outcopies[p] = ocp
        for p in range(nbuf):
            if outcopies[p] is not None:
                outcopies[p].wait()
        plsc.subcore_barrier()

        @pl.when(s == 0)
        def _scatter():
            pltpu.sync_copy(idx_hbm, idx_v)
            row0 = c * rows_per_core
            for j in range(rows_per_core // 16):
                rid = (lax.iota(jnp.int32, 16) + (row0 + j * 16)) * S + idx_v[...]
                rowid_v[pl.ds(j * 16, 16)] = rid
            xr = buf0.at[pl.ds(0, rows_per_core)]
            pltpu.sync_copy(x_hbm.at[pl.ds(row0, rows_per_core)], xr)
            sc = pltpu.async_copy(xr, o_hbm.at[rowid_v], sem_s)
            sc.wait()

    return sc_kernel(cache2, x2, idxv)


def kernel(k_cache, v_cache, xk, xv, layer_past_len):
    B, Hkv, S, D = k_cache.shape
    N = B * Hkv
    idx = jnp.asarray(layer_past_len, jnp.int32).reshape((1,))
    idxv = jnp.full((16,), jnp.asarray(layer_past_len, jnp.int32))
    with compute_on("tpu_sparsecore"):
        ov = _sc_update(v_cache.reshape(N * S, D), xv.reshape(N, D), idxv, N, S, D)
    ok = _tc_update(k_cache.reshape(N, S, D), xk.reshape(N, 1, D), idx)
    return ok.reshape(B, Hkv, S, D), ov.reshape(B, Hkv, S, D)


# speedup vs baseline: 1.0389x; 1.0389x over previous
import functools

import jax
import jax.numpy as jnp
from jax import lax
from jax.experimental import pallas as pl
from jax.experimental.pallas import tpu as pltpu
from jax.experimental.pallas import tpu_sc as plsc

_NC = 2
_NS = 16
_G = 8


def _tc_copy_body(c_ref, o_ref):
    o_ref[...] = c_ref[...]


def _tc_copy(cache2, rows_per_block):
    M, D = cache2.shape
    spec = pl.BlockSpec((rows_per_block, D), lambda i: (i, 0))
    return pl.pallas_call(
        _tc_copy_body,
        grid=(M // rows_per_block,),
        in_specs=[spec],
        out_specs=spec,
        out_shape=jax.ShapeDtypeStruct(cache2.shape, cache2.dtype),
    )(cache2)


def _sc_scatter(ok_ref, ov_ref, xk2, xv2, idxv, N, S, D):
    rows_per_core = N // _NC
    mesh = plsc.VectorSubcoreMesh(
        core_axis_name="c", subcore_axis_name="s", num_cores=_NC, num_subcores=_NS
    )

    @functools.partial(
        pl.kernel,
        out_type=(),
        mesh=mesh,
        scratch_types=[
            pltpu.VMEM((16,), jnp.int32),
            pltpu.VMEM((rows_per_core,), jnp.int32),
            pltpu.VMEM((rows_per_core, D), jnp.float32),
            pltpu.VMEM((rows_per_core, D), jnp.float32),
            pltpu.SemaphoreType.DMA,
            pltpu.SemaphoreType.DMA,
        ],
    )
    def sck(xk_hbm, xv_hbm, idx_hbm, ok_hbm, ov_hbm,
            idx_v, rowid_v, xkr, xvr, sk, sv):
        c = lax.axis_index("c")
        s = lax.axis_index("s")

        @pl.when(s == 0)
        def _scatter():
            pltpu.sync_copy(idx_hbm, idx_v)
            row0 = c * rows_per_core
            for j in range(rows_per_core // 16):
                rid = (lax.iota(jnp.int32, 16) + (row0 + j * 16)) * S + idx_v[...]
                rowid_v[pl.ds(j * 16, 16)] = rid
            pltpu.sync_copy(xk_hbm.at[pl.ds(row0, rows_per_core)], xkr)
            pltpu.sync_copy(xv_hbm.at[pl.ds(row0, rows_per_core)], xvr)
            a = pltpu.async_copy(xkr, ok_hbm.at[rowid_v], sk)
            b = pltpu.async_copy(xvr, ov_hbm.at[rowid_v], sv)
            a.wait()
            b.wait()

    sck(xk2, xv2, idxv, ok_ref, ov_ref)


def kernel(k_cache, v_cache, xk, xv, layer_past_len):
    B, Hkv, S, D = k_cache.shape
    N = B * Hkv
    idxv = jnp.full((16,), jnp.asarray(layer_past_len, jnp.int32))
    ok2 = _tc_copy(k_cache.reshape(N * S, D), _G * S)
    ov2 = _tc_copy(v_cache.reshape(N * S, D), _G * S)
    ok_ref = jax.new_ref(ok2)
    ov_ref = jax.new_ref(ov2)
    _sc_scatter(ok_ref, ov_ref, xk.reshape(N, D), xv.reshape(N, D), idxv, N, S, D)
    ok = jax.freeze(ok_ref)
    ov = jax.freeze(ov_ref)
    return ok.reshape(B, Hkv, S, D), ov.reshape(B, Hkv, S, D)


# speedup vs baseline: 1.0465x; 1.0073x over previous
import functools

import jax
import jax.numpy as jnp
from jax import lax
from jax.experimental import pallas as pl
from jax.experimental.pallas import tpu as pltpu
from jax.experimental.pallas import tpu_sc as plsc

_NC = 2
_NS = 16
_G = 8


def _tc_copy_body(c_ref, o_ref):
    o_ref[...] = c_ref[...]


def _tc_copy(cache2, rows_per_block):
    M, D = cache2.shape
    spec = pl.BlockSpec((rows_per_block, D), lambda i: (i, 0))
    return pl.pallas_call(
        _tc_copy_body,
        grid=(M // rows_per_block,),
        in_specs=[spec],
        out_specs=spec,
        out_shape=jax.ShapeDtypeStruct(cache2.shape, cache2.dtype),
    )(cache2)


def _sc_scatter(ok_ref, ov_ref, xk2, xv2, idxv, N, S, D):
    mesh = plsc.VectorSubcoreMesh(
        core_axis_name="c", subcore_axis_name="s", num_cores=_NC, num_subcores=_NS
    )

    @functools.partial(
        pl.kernel,
        out_type=(),
        mesh=mesh,
        scratch_types=[
            pltpu.VMEM((16,), jnp.int32),
            pltpu.VMEM((N,), jnp.int32),
            pltpu.VMEM((N, D), jnp.float32),
            pltpu.SemaphoreType.DMA,
            pltpu.SemaphoreType.DMA,
        ],
    )
    def sck(xk_hbm, xv_hbm, idx_hbm, ok_hbm, ov_hbm,
            idx_v, rowid_v, xr, s_stage, s_scatter):
        c = lax.axis_index("c")
        s = lax.axis_index("s")

        @pl.when(s == 0)
        def _scatter():
            x_hbm = [xk_hbm, xv_hbm]
            o_hbm = [ok_hbm, ov_hbm]
            stages = [
                pltpu.make_async_copy(x_hbm[cc], xr, s_stage) for cc in range(_NC)
            ]
            idx_cp = pltpu.make_async_copy(idx_hbm, idx_v, s_scatter)
            idx_cp.start()
            for cc in range(_NC):
                @pl.when(c == cc)
                def _start_stage(cc=cc):
                    stages[cc].start()
            idx_cp.wait()
            for j in range(N // 16):
                rid = (lax.iota(jnp.int32, 16) + j * 16) * S + idx_v[...]
                rowid_v[pl.ds(j * 16, 16)] = rid
            scatters = [
                pltpu.make_async_copy(xr, o_hbm[cc].at[rowid_v], s_scatter)
                for cc in range(_NC)
            ]
            for cc in range(_NC):
                @pl.when(c == cc)
                def _finish(cc=cc):
                    stages[cc].wait()
                    scatters[cc].start()
                    scatters[cc].wait()

    sck(xk2, xv2, idxv, ok_ref, ov_ref)


def kernel(k_cache, v_cache, xk, xv, layer_past_len):
    B, Hkv, S, D = k_cache.shape
    N = B * Hkv
    idxv = jnp.full((16,), jnp.asarray(layer_past_len, jnp.int32))
    ok2 = _tc_copy(k_cache.reshape(N * S, D), _G * S)
    ov2 = _tc_copy(v_cache.reshape(N * S, D), _G * S)
    ok_ref = jax.new_ref(ok2)
    ov_ref = jax.new_ref(ov2)
    _sc_scatter(ok_ref, ov_ref, xk.reshape(N, D), xv.reshape(N, D), idxv, N, S, D)
    ok = jax.freeze(ok_ref)
    ov = jax.freeze(ov_ref)
    return ok.reshape(B, Hkv, S, D), ov.reshape(B, Hkv, S, D)


# speedup vs baseline: 1.0492x; 1.0026x over previous
import functools

import jax
import jax.numpy as jnp
from jax import lax
from jax.experimental import pallas as pl
from jax.experimental.pallas import tpu as pltpu
from jax.experimental.pallas import tpu_sc as plsc

_NC = 2
_NS = 16
_G = 8


def _tc_copy_body(c_ref, o_ref):
    o_ref[...] = c_ref[...]


def _tc_copy(cache2, rows_per_block):
    M, D = cache2.shape
    spec = pl.BlockSpec((rows_per_block, D), lambda i: (i, 0))
    return pl.pallas_call(
        _tc_copy_body,
        grid=(M // rows_per_block,),
        in_specs=[spec],
        out_specs=spec,
        out_shape=jax.ShapeDtypeStruct(cache2.shape, cache2.dtype),
    )(cache2)


def _sc_scatter(o_ref, x2, idxv, N, S, D):
    rows_per_core = N // _NC
    mesh = plsc.VectorSubcoreMesh(
        core_axis_name="c", subcore_axis_name="s", num_cores=_NC, num_subcores=_NS
    )

    @functools.partial(
        pl.kernel,
        out_type=(),
        mesh=mesh,
        scratch_types=[
            pltpu.VMEM((16,), jnp.int32),
            pltpu.VMEM((rows_per_core,), jnp.int32),
            pltpu.VMEM((rows_per_core, D), jnp.float32),
            pltpu.SemaphoreType.DMA,
            pltpu.SemaphoreType.DMA,
        ],
    )
    def sck(x_hbm, idx_hbm, o_hbm, idx_v, rowid_v, xr, s_stage, s_scatter):
        c = lax.axis_index("c")
        s = lax.axis_index("s")

        @pl.when(s == 0)
        def _scatter():
            row0 = c * rows_per_core
            stage = pltpu.make_async_copy(
                x_hbm.at[pl.ds(row0, rows_per_core)], xr, s_stage
            )
            stage.start()
            idx_cp = pltpu.make_async_copy(idx_hbm, idx_v, s_scatter)
            idx_cp.start()
            idx_cp.wait()
            for j in range(rows_per_core // 16):
                rid = (lax.iota(jnp.int32, 16) + (row0 + j * 16)) * S + idx_v[...]
                rowid_v[pl.ds(j * 16, 16)] = rid
            stage.wait()
            sc = pltpu.async_copy(xr, o_hbm.at[rowid_v], s_scatter)
            sc.wait()

    sck(x2, idxv, o_ref)


def kernel(k_cache, v_cache, xk, xv, layer_past_len):
    B, Hkv, S, D = k_cache.shape
    N = B * Hkv
    idxv = jnp.full((16,), jnp.asarray(layer_past_len, jnp.int32))
    ok2 = _tc_copy(k_cache.reshape(N * S, D), _G * S)
    ok_ref = jax.new_ref(ok2)
    _sc_scatter(ok_ref, xk.reshape(N, D), idxv, N, S, D)
    ov2 = _tc_copy(v_cache.reshape(N * S, D), _G * S)
    ov_ref = jax.new_ref(ov2)
    _sc_scatter(ov_ref, xv.reshape(N, D), idxv, N, S, D)
    ok = jax.freeze(ok_ref)
    ov = jax.freeze(ov_ref)
    return ok.reshape(B, Hkv, S, D), ov.reshape(B, Hkv, S, D)
